# Initial kernel scaffold; baseline (speedup 1.0000x reference)
import jax
import jax.numpy as jnp
from jax.experimental import pallas as pl


def kernel(x, edge_index, W_conv, b_conv, H, W):
    raise NotImplementedError("write your pallas kernel here")


# speedup vs baseline: 1939.8406x; 1939.8406x over previous
import functools

import numpy as np
import jax
import jax.numpy as jnp
from jax import lax
from jax.experimental import pallas as pl
from jax.experimental.pallas import tpu as pltpu
from jax.experimental.pallas import tpu_sc as plsc

_B, _C, _N, _K = 4, 96, 2304, 16
_NCENT, _TOPK, _G = 50, 12, 4
_PREC = lax.Precision.HIGHEST

_NTILES = 32
_NPT = (_B * _N) // _NTILES
_NB = 16
_NCHUNK = _NPT // _NB
_ROWS = _NB * 2 * _K
_IDXCHUNK = 128
_NGATHER = _ROWS // _IDXCHUNK
_LANES = 16
_CC = _C // _LANES
_CPAD = 128


def _pool_matrix() -> np.ndarray:
    Hs = Ws = 48
    oh, ow = 5, 10
    M = np.zeros((Hs * Ws, oh * ow), np.float32)
    for i in range(oh):
        h0 = (i * Hs) // oh
        h1 = -(-((i + 1) * Hs) // oh)
        for j in range(ow):
            w0 = (j * Ws) // ow
            w1 = -(-((j + 1) * Ws) // ow)
            cnt = (h1 - h0) * (w1 - w0)
            for h in range(h0, h1):
                for w in range(w0, w1):
                    M[h * Ws + w, i * ow + j] = 1.0 / cnt
    return M


_M_POOL = _pool_matrix()
_GMASK = ((np.arange(_C)[:, None] // (_C // _G))
          == (np.arange(_C)[None, :] // (_C // _G))).astype(np.float32)


def _sc_edge_max_body(x_hbm, e0_hbm, e1_hbm, out_hbm, idx_v, e0_v, e1_v, rows_v, y_v, sem):
    wid = lax.axis_index("s") * 2 + lax.axis_index("c")
    base = wid * _NPT
    boff = (base // _N) * _N

    def chunk(ch, carry):
        g0 = base + ch * _NB
        pltpu.sync_copy(e0_hbm.at[pl.ds(g0 * _K, _NB * _K)], e0_v)
        pltpu.sync_copy(e1_hbm.at[pl.ds(g0 * _K, _NB * _K)], e1_v)

        def build(i, c2):
            sl = pl.ds(i * _K, _K)
            idx_v[pl.ds(i * 2 * _K, _K)] = e0_v[sl] + boff
            idx_v[pl.ds(i * 2 * _K + _K, _K)] = e1_v[sl] + boff
            return c2

        lax.fori_loop(0, _NB, build, 0)
        handles = [
            pltpu.async_copy(
                x_hbm.at[idx_v.at[pl.ds(r * _IDXCHUNK, _IDXCHUNK)]],
                rows_v.at[pl.ds(r * _IDXCHUNK, _IDXCHUNK)],
                sem,
            )
            for r in range(_NGATHER)
        ]
        for h in handles:
            h.wait()

        def comp(i, c2):
            r0 = i * 2 * _K
            for cc in range(_CC):
                sl = pl.ds(cc * _LANES, _LANES)
                acc = rows_v[r0, sl] - rows_v[r0 + _K, sl]
                for k in range(1, _K):
                    acc = jnp.maximum(acc, rows_v[r0 + k, sl] - rows_v[r0 + _K + k, sl])
                y_v[pl.ds(i * _C + cc * _LANES, _LANES)] = acc
            return c2

        lax.fori_loop(0, _NB, comp, 0)
        pltpu.sync_copy(y_v, out_hbm.at[pl.ds(g0 * _C, _NB * _C)])
        return carry

    lax.fori_loop(0, _NCHUNK, chunk, 0)


@functools.cache
def _sc_edge_max():
    return pl.kernel(
        _sc_edge_max_body,
        out_type=jax.ShapeDtypeStruct((_B * _N * _C,), jnp.float32),
        mesh=plsc.VectorSubcoreMesh(core_axis_name="c", subcore_axis_name="s"),
        scratch_types=[
            pltpu.VMEM((_ROWS,), jnp.int32),
            pltpu.VMEM((_NB * _K,), jnp.int32),
            pltpu.VMEM((_NB * _K,), jnp.int32),
            pltpu.VMEM((_ROWS, _CPAD), jnp.float32),
            pltpu.VMEM((_NB * _C,), jnp.float32),
            pltpu.SemaphoreType.DMA,
        ],
    )


def _tc_body(x_ref, xt_ref, ye_ref, m_ref, wxc_ref, we_ref, wc_ref, b_ref, o_ref):
    xb = x_ref[0]
    xt = xt_ref[0]
    ye = ye_ref[0]
    cent = jnp.dot(xb, m_ref[...], precision=_PREC)
    w = None
    for _ in range(3):
        xc = jnp.dot(xt, cent, precision=_PREC)
        c2 = jnp.sum(cent * cent, axis=0)[None, :]
        logits = 2.0 * xc - c2
        mx = jnp.max(logits, axis=1, keepdims=True)
        e = jnp.exp(logits - mx)
        w = e / jnp.sum(e, axis=1, keepdims=True)
        denom = jnp.sum(w, axis=0)[None, :] + 1e-8
        cent = jnp.dot(xb, w, precision=_PREC) / denom

    iota = lax.broadcasted_iota(jnp.int32, (_N, _NCENT), 1)
    avail = jnp.ones((_N, _NCENT), jnp.bool_)
    acc = None
    for _ in range(_TOPK):
        cur = jnp.where(avail, w, -1.0)
        mx = jnp.max(cur, axis=1, keepdims=True)
        ism = cur == mx
        minj = jnp.min(jnp.where(ism, iota, _NCENT), axis=1, keepdims=True)
        sel = iota == minj
        pick = lax.dot_general(sel.astype(jnp.float32), cent,
                               (((1,), (1,)), ((), ())), precision=_PREC)
        acc = pick if acc is None else jnp.maximum(acc, pick)
        avail = avail & jnp.logical_not(sel)

    out = lax.dot_general(xt, wxc_ref[...], (((1,), (1,)), ((), ())),
                          precision=_PREC)
    out = out + lax.dot_general(ye, we_ref[...], (((1,), (1,)), ((), ())),
                                precision=_PREC)
    out = out + lax.dot_general(acc, wc_ref[...], (((1,), (1,)), ((), ())),
                                precision=_PREC)
    out = out + b_ref[...]
    o_ref[0] = jnp.maximum(out, 0.0)


def kernel(x, edge_index, W_conv, b_conv, H, W):
    xb = x[..., 0]
    xt = jnp.swapaxes(xb, 1, 2)
    x_rows = jnp.pad(xt.reshape(_B * _N, _C), ((0, 0), (0, _CPAD - _C)))
    ei = edge_index.astype(jnp.int32)
    e0 = ei[0].reshape(-1)
    e1 = ei[1].reshape(-1)
    y_edge = _sc_edge_max()(x_rows, e0, e1)
    ye = y_edge.reshape(_B, _N, _C)

    w3 = W_conv[:, :, 0, 0].reshape(_C, _C // _G, 3)
    gmask = jnp.asarray(_GMASK)

    def expand(ws):
        return jnp.tile(ws, (1, _G)) * gmask

    Wx = expand(w3[:, :, 0])
    We = expand(w3[:, :, 1])
    Wc = expand(w3[:, :, 2])
    Wxc = Wx - Wc
    b2 = b_conv.reshape(1, _C)

    out = pl.pallas_call(
        _tc_body,
        grid=(_B,),
        in_specs=[
            pl.BlockSpec((1, _C, _N), lambda b: (b, 0, 0)),
            pl.BlockSpec((1, _N, _C), lambda b: (b, 0, 0)),
            pl.BlockSpec((1, _N, _C), lambda b: (b, 0, 0)),
            pl.BlockSpec((_N, _NCENT), lambda b: (0, 0)),
            pl.BlockSpec((_C, _C), lambda b: (0, 0)),
            pl.BlockSpec((_C, _C), lambda b: (0, 0)),
            pl.BlockSpec((_C, _C), lambda b: (0, 0)),
            pl.BlockSpec((1, _C), lambda b: (0, 0)),
        ],
        out_specs=pl.BlockSpec((1, _N, _C), lambda b: (b, 0, 0)),
        out_shape=jax.ShapeDtypeStruct((_B, _N, _C), jnp.float32),
    )(xb, xt, ye, jnp.asarray(_M_POOL), Wxc, We, Wc, b2)
    return jnp.swapaxes(out, 1, 2)[..., None]


# speedup vs baseline: 2631.6124x; 1.3566x over previous
import functools

import numpy as np
import jax
import jax.numpy as jnp
from jax import lax
from jax.experimental import pallas as pl
from jax.experimental.pallas import tpu as pltpu
from jax.experimental.pallas import tpu_sc as plsc

_B, _C, _N, _K = 4, 96, 2304, 16
_NCENT, _TOPK, _G = 50, 12, 4
_PREC = None

_NTILES = 32
_NPT = (_B * _N) // _NTILES
_NB = 16
_NCHUNK = _NPT // _NB
_ROWS = _NB * 2 * _K
_IDXCHUNK = 128
_NGATHER = _ROWS // _IDXCHUNK
_LANES = 16
_CC = _C // _LANES
_CPAD = 128


def _pool_matrix() -> np.ndarray:
    Hs = Ws = 48
    oh, ow = 5, 10
    M = np.zeros((Hs * Ws, oh * ow), np.float32)
    for i in range(oh):
        h0 = (i * Hs) // oh
        h1 = -(-((i + 1) * Hs) // oh)
        for j in range(ow):
            w0 = (j * Ws) // ow
            w1 = -(-((j + 1) * Ws) // ow)
            cnt = (h1 - h0) * (w1 - w0)
            for h in range(h0, h1):
                for w in range(w0, w1):
                    M[h * Ws + w, i * ow + j] = 1.0 / cnt
    return M


_M_POOL = _pool_matrix()
_GMASK = ((np.arange(_C)[:, None] // (_C // _G))
          == (np.arange(_C)[None, :] // (_C // _G))).astype(np.float32)


def _sc_edge_max_body(x_hbm, e0_hbm, e1_hbm, out_hbm, idx_v, e0_v, e1_v, rows_v, y_v, sem):
    wid = lax.axis_index("s") * 2 + lax.axis_index("c")
    base = wid * _NPT
    boff = (base // _N) * _N

    def chunk(ch, carry):
        g0 = base + ch * _NB
        pltpu.sync_copy(e0_hbm.at[pl.ds(g0 * _K, _NB * _K)], e0_v)
        pltpu.sync_copy(e1_hbm.at[pl.ds(g0 * _K, _NB * _K)], e1_v)

        def build(i, c2):
            sl = pl.ds(i * _K, _K)
            idx_v[pl.ds(i * 2 * _K, _K)] = e0_v[sl] + boff
            idx_v[pl.ds(i * 2 * _K + _K, _K)] = e1_v[sl] + boff
            return c2

        lax.fori_loop(0, _NB, build, 0)
        handles = [
            pltpu.async_copy(
                x_hbm.at[idx_v.at[pl.ds(r * _IDXCHUNK, _IDXCHUNK)]],
                rows_v.at[pl.ds(r * _IDXCHUNK, _IDXCHUNK)],
                sem,
            )
            for r in range(_NGATHER)
        ]
        for h in handles:
            h.wait()

        def comp(i, c2):
            r0 = i * 2 * _K
            for cc in range(_CC):
                sl = pl.ds(cc * _LANES, _LANES)
                acc = rows_v[r0, sl] - rows_v[r0 + _K, sl]
                for k in range(1, _K):
                    acc = jnp.maximum(acc, rows_v[r0 + k, sl] - rows_v[r0 + _K + k, sl])
                y_v[pl.ds(i * _C + cc * _LANES, _LANES)] = acc
            return c2

        lax.fori_loop(0, _NB, comp, 0)
        pltpu.sync_copy(y_v, out_hbm.at[pl.ds(g0 * _C, _NB * _C)])
        return carry

    lax.fori_loop(0, _NCHUNK, chunk, 0)


@functools.cache
def _sc_edge_max():
    return pl.kernel(
        _sc_edge_max_body,
        out_type=jax.ShapeDtypeStruct((_B * _N * _C,), jnp.float32),
        mesh=plsc.VectorSubcoreMesh(core_axis_name="c", subcore_axis_name="s"),
        scratch_types=[
            pltpu.VMEM((_ROWS,), jnp.int32),
            pltpu.VMEM((_NB * _K,), jnp.int32),
            pltpu.VMEM((_NB * _K,), jnp.int32),
            pltpu.VMEM((_ROWS, _CPAD), jnp.float32),
            pltpu.VMEM((_NB * _C,), jnp.float32),
            pltpu.SemaphoreType.DMA,
        ],
    )


def _tc_body(x_ref, xt_ref, ye_ref, m_ref, wxc_ref, we_ref, wc_ref, b_ref, o_ref):
    xb = x_ref[0]
    xt = xt_ref[0]
    ye = ye_ref[0]
    cent = jnp.dot(xb, m_ref[...], precision=_PREC)
    w = None
    for _ in range(3):
        xc = jnp.dot(xt, cent, precision=_PREC)
        c2 = jnp.sum(cent * cent, axis=0)[None, :]
        logits = 2.0 * xc - c2
        mx = jnp.max(logits, axis=1, keepdims=True)
        e = jnp.exp(logits - mx)
        w = e / jnp.sum(e, axis=1, keepdims=True)
        denom = jnp.sum(w, axis=0)[None, :] + 1e-8
        cent = jnp.dot(xb, w, precision=_PREC) / denom

    iota = lax.broadcasted_iota(jnp.int32, (_N, _NCENT), 1)
    avail = jnp.ones((_N, _NCENT), jnp.bool_)
    acc = None
    for _ in range(_TOPK):
        cur = jnp.where(avail, w, -1.0)
        mx = jnp.max(cur, axis=1, keepdims=True)
        ism = cur == mx
        minj = jnp.min(jnp.where(ism, iota, _NCENT), axis=1, keepdims=True)
        sel = iota == minj
        pick = lax.dot_general(sel.astype(jnp.float32), cent,
                               (((1,), (1,)), ((), ())), precision=_PREC)
        acc = pick if acc is None else jnp.maximum(acc, pick)
        avail = avail & jnp.logical_not(sel)

    out = lax.dot_general(xt, wxc_ref[...], (((1,), (1,)), ((), ())),
                          precision=_PREC)
    out = out + lax.dot_general(ye, we_ref[...], (((1,), (1,)), ((), ())),
                                precision=_PREC)
    out = out + lax.dot_general(acc, wc_ref[...], (((1,), (1,)), ((), ())),
                                precision=_PREC)
    out = out + b_ref[...]
    o_ref[0] = jnp.maximum(out, 0.0)


def kernel(x, edge_index, W_conv, b_conv, H, W):
    xb = x[..., 0]
    xt = jnp.swapaxes(xb, 1, 2)
    x_rows = jnp.pad(xt.reshape(_B * _N, _C), ((0, 0), (0, _CPAD - _C)))
    ei = edge_index.astype(jnp.int32)
    e0 = ei[0].reshape(-1)
    e1 = ei[1].reshape(-1)
    y_edge = _sc_edge_max()(x_rows, e0, e1)
    ye = y_edge.reshape(_B, _N, _C)

    w3 = W_conv[:, :, 0, 0].reshape(_C, _C // _G, 3)
    gmask = jnp.asarray(_GMASK)

    def expand(ws):
        return jnp.tile(ws, (1, _G)) * gmask

    Wx = expand(w3[:, :, 0])
    We = expand(w3[:, :, 1])
    Wc = expand(w3[:, :, 2])
    Wxc = Wx - Wc
    b2 = b_conv.reshape(1, _C)

    out = pl.pallas_call(
        _tc_body,
        grid=(_B,),
        in_specs=[
            pl.BlockSpec((1, _C, _N), lambda b: (b, 0, 0)),
            pl.BlockSpec((1, _N, _C), lambda b: (b, 0, 0)),
            pl.BlockSpec((1, _N, _C), lambda b: (b, 0, 0)),
            pl.BlockSpec((_N, _NCENT), lambda b: (0, 0)),
            pl.BlockSpec((_C, _C), lambda b: (0, 0)),
            pl.BlockSpec((_C, _C), lambda b: (0, 0)),
            pl.BlockSpec((_C, _C), lambda b: (0, 0)),
            pl.BlockSpec((1, _C), lambda b: (0, 0)),
        ],
        out_specs=pl.BlockSpec((1, _N, _C), lambda b: (b, 0, 0)),
        out_shape=jax.ShapeDtypeStruct((_B, _N, _C), jnp.float32),
    )(xb, xt, ye, jnp.asarray(_M_POOL), Wxc, We, Wc, b2)
    return jnp.swapaxes(out, 1, 2)[..., None]


# speedup vs baseline: 3510.6241x; 1.3340x over previous
import functools

import numpy as np
import jax
import jax.numpy as jnp
from jax import lax
from jax.experimental import pallas as pl
from jax.experimental.pallas import tpu as pltpu
from jax.experimental.pallas import tpu_sc as plsc

_B, _C, _N, _K = 4, 96, 2304, 16
_NCENT, _TOPK, _G = 50, 12, 4
_PREC = None

_NTILES = 32
_NPT = (_B * _N) // _NTILES
_NB = 16
_NCHUNK = _NPT // _NB
_ROWS = _NB * 2 * _K
_IDXCHUNK = 128
_NGATHER = _ROWS // _IDXCHUNK
_LANES = 16
_CC = _C // _LANES


def _pool_matrix() -> np.ndarray:
    Hs = Ws = 48
    oh, ow = 5, 10
    M = np.zeros((Hs * Ws, oh * ow), np.float32)
    for i in range(oh):
        h0 = (i * Hs) // oh
        h1 = -(-((i + 1) * Hs) // oh)
        for j in range(ow):
            w0 = (j * Ws) // ow
            w1 = -(-((j + 1) * Ws) // ow)
            cnt = (h1 - h0) * (w1 - w0)
            for h in range(h0, h1):
                for w in range(w0, w1):
                    M[h * Ws + w, i * ow + j] = 1.0 / cnt
    return M


_M_POOL = _pool_matrix()
_GMASK = ((np.arange(_C)[:, None] // (_C // _G))
          == (np.arange(_C)[None, :] // (_C // _G))).astype(np.float32)


def _sc_edge_max_body(x_hbm, e0_hbm, e1_hbm, out_hbm, idx_v, e0_v, e1_v, rows_v, y_v,
                      sem0, sem1):
    wid = lax.axis_index("s") * 2 + lax.axis_index("c")
    base = wid * _NPT
    boff = (base // _N) * _N
    sems = (sem0, sem1)

    def load_fire(ch, buf):
        g0 = base + ch * _NB
        pltpu.sync_copy(e0_hbm.at[pl.ds(g0 * _K, _NB * _K)], e0_v)
        pltpu.sync_copy(e1_hbm.at[pl.ds(g0 * _K, _NB * _K)], e1_v)
        o = buf * _ROWS

        def build(i, c2):
            sl = pl.ds(i * _K, _K)
            idx_v[pl.ds(o + i * 2 * _K, _K)] = e0_v[sl] + boff
            idx_v[pl.ds(o + i * 2 * _K + _K, _K)] = e1_v[sl] + boff
            return c2

        lax.fori_loop(0, _NB, build, 0)
        for r in range(_NGATHER):
            pltpu.async_copy(
                x_hbm.at[idx_v.at[pl.ds(o + r * _IDXCHUNK, _IDXCHUNK)]],
                rows_v.at[pl.ds(o + r * _IDXCHUNK, _IDXCHUNK)],
                sems[buf],
            )

    def wait_gathers(buf):
        o = buf * _ROWS
        for r in range(_NGATHER):
            pltpu.make_async_copy(
                x_hbm.at[idx_v.at[pl.ds(o + r * _IDXCHUNK, _IDXCHUNK)]],
                rows_v.at[pl.ds(o + r * _IDXCHUNK, _IDXCHUNK)],
                sems[buf],
            ).wait()

    def compute_store(ch, buf):
        g0 = base + ch * _NB
        o = buf * _ROWS

        def comp(i, c2):
            r0 = o + i * 2 * _K
            for cc in range(_CC):
                sl = pl.ds(cc * _LANES, _LANES)
                acc = rows_v[r0, sl] - rows_v[r0 + _K, sl]
                for k in range(1, _K):
                    acc = jnp.maximum(acc, rows_v[r0 + k, sl] - rows_v[r0 + _K + k, sl])
                y_v[pl.ds(i * _C + cc * _LANES, _LANES)] = acc
            return c2

        lax.fori_loop(0, _NB, comp, 0)
        pltpu.sync_copy(y_v, out_hbm.at[pl.ds(g0 * _C, _NB * _C)])

    load_fire(0, 0)

    def outer(g, carry):
        for b in (0, 1):
            ch = g * 2 + b

            @pl.when(ch + 1 < _NCHUNK)
            def _():
                load_fire(ch + 1, b ^ 1)

            wait_gathers(b)
            compute_store(ch, b)
        return carry

    lax.fori_loop(0, _NCHUNK // 2, outer, 0)


@functools.cache
def _sc_edge_max():
    return pl.kernel(
        _sc_edge_max_body,
        out_type=jax.ShapeDtypeStruct((_B * _N * _C,), jnp.float32),
        mesh=plsc.VectorSubcoreMesh(core_axis_name="c", subcore_axis_name="s"),
        compiler_params=pltpu.CompilerParams(use_tc_tiling_on_sc=False),
        scratch_types=[
            pltpu.VMEM((2 * _ROWS,), jnp.int32),
            pltpu.VMEM((_NB * _K,), jnp.int32),
            pltpu.VMEM((_NB * _K,), jnp.int32),
            pltpu.VMEM((2 * _ROWS, _C), jnp.float32),
            pltpu.VMEM((_NB * _C,), jnp.float32),
            pltpu.SemaphoreType.DMA,
            pltpu.SemaphoreType.DMA,
        ],
    )


def _tc_body(x_ref, xt_ref, ye_ref, m_ref, wxc_ref, we_ref, wc_ref, b_ref, o_ref):
    xb = x_ref[0]
    xt = xt_ref[0]
    ye = ye_ref[0]
    cent = jnp.dot(xb, m_ref[...], precision=_PREC)
    w = None
    for _ in range(3):
        xc = jnp.dot(xt, cent, precision=_PREC)
        c2 = jnp.sum(cent * cent, axis=0)[None, :]
        logits = 2.0 * xc - c2
        mx = jnp.max(logits, axis=1, keepdims=True)
        e = jnp.exp(logits - mx)
        w = e / jnp.sum(e, axis=1, keepdims=True)
        denom = jnp.sum(w, axis=0)[None, :] + 1e-8
        cent = jnp.dot(xb, w, precision=_PREC) / denom

    iota = lax.broadcasted_iota(jnp.int32, (_N, _NCENT), 1)
    avail = jnp.ones((_N, _NCENT), jnp.bool_)
    acc = None
    for _ in range(_TOPK):
        cur = jnp.where(avail, w, -1.0)
        mx = jnp.max(cur, axis=1, keepdims=True)
        ism = cur == mx
        minj = jnp.min(jnp.where(ism, iota, _NCENT), axis=1, keepdims=True)
        sel = iota == minj
        pick = lax.dot_general(sel.astype(jnp.float32), cent,
                               (((1,), (1,)), ((), ())), precision=_PREC)
        acc = pick if acc is None else jnp.maximum(acc, pick)
        avail = avail & jnp.logical_not(sel)

    out = lax.dot_general(xt, wxc_ref[...], (((1,), (1,)), ((), ())),
                          precision=_PREC)
    out = out + lax.dot_general(ye, we_ref[...], (((1,), (1,)), ((), ())),
                                precision=_PREC)
    out = out + lax.dot_general(acc, wc_ref[...], (((1,), (1,)), ((), ())),
                                precision=_PREC)
    out = out + b_ref[...]
    o_ref[0] = jnp.maximum(out, 0.0)


def kernel(x, edge_index, W_conv, b_conv, H, W):
    xb = x[..., 0]
    xt = jnp.swapaxes(xb, 1, 2)
    x_rows = xt.reshape(_B * _N, _C)
    ei = edge_index.astype(jnp.int32)
    e0 = ei[0].reshape(-1)
    e1 = ei[1].reshape(-1)
    y_edge = _sc_edge_max()(x_rows, e0, e1)
    ye = y_edge.reshape(_B, _N, _C)

    w3 = W_conv[:, :, 0, 0].reshape(_C, _C // _G, 3)
    gmask = jnp.asarray(_GMASK)

    def expand(ws):
        return jnp.tile(ws, (1, _G)) * gmask

    Wx = expand(w3[:, :, 0])
    We = expand(w3[:, :, 1])
    Wc = expand(w3[:, :, 2])
    Wxc = Wx - Wc
    b2 = b_conv.reshape(1, _C)

    out = pl.pallas_call(
        _tc_body,
        grid=(_B,),
        in_specs=[
            pl.BlockSpec((1, _C, _N), lambda b: (b, 0, 0)),
            pl.BlockSpec((1, _N, _C), lambda b: (b, 0, 0)),
            pl.BlockSpec((1, _N, _C), lambda b: (b, 0, 0)),
            pl.BlockSpec((_N, _NCENT), lambda b: (0, 0)),
            pl.BlockSpec((_C, _C), lambda b: (0, 0)),
            pl.BlockSpec((_C, _C), lambda b: (0, 0)),
            pl.BlockSpec((_C, _C), lambda b: (0, 0)),
            pl.BlockSpec((1, _C), lambda b: (0, 0)),
        ],
        out_specs=pl.BlockSpec((1, _N, _C), lambda b: (b, 0, 0)),
        out_shape=jax.ShapeDtypeStruct((_B, _N, _C), jnp.float32),
    )(xb, xt, ye, jnp.asarray(_M_POOL), Wxc, We, Wc, b2)
    return jnp.swapaxes(out, 1, 2)[..., None]


# speedup vs baseline: 3774.0735x; 1.0750x over previous
import functools

import numpy as np
import jax
import jax.numpy as jnp
from jax import lax
from jax.experimental import pallas as pl
from jax.experimental.pallas import tpu as pltpu
from jax.experimental.pallas import tpu_sc as plsc

_B, _C, _N, _K = 4, 96, 2304, 16
_NCENT, _TOPK, _G = 50, 12, 4
_PREC = None

_NTILES = 32
_NPT = (_B * _N) // _NTILES
_NB = 16
_NCHUNK = _NPT // _NB
_ROWS = _NB * 2 * _K
_IDXCHUNK = 128
_NGATHER = _ROWS // _IDXCHUNK
_LANES = 16
_BLANES = 32
_CCB = _C // _BLANES


def _pool_matrix() -> np.ndarray:
    Hs = Ws = 48
    oh, ow = 5, 10
    M = np.zeros((Hs * Ws, oh * ow), np.float32)
    for i in range(oh):
        h0 = (i * Hs) // oh
        h1 = -(-((i + 1) * Hs) // oh)
        for j in range(ow):
            w0 = (j * Ws) // ow
            w1 = -(-((j + 1) * Ws) // ow)
            cnt = (h1 - h0) * (w1 - w0)
            for h in range(h0, h1):
                for w in range(w0, w1):
                    M[h * Ws + w, i * ow + j] = 1.0 / cnt
    return M


_M_POOL = _pool_matrix()
_GMASK = ((np.arange(_C)[:, None] // (_C // _G))
          == (np.arange(_C)[None, :] // (_C // _G))).astype(np.float32)


def _sc_edge_max_body(x_hbm, e0_hbm, e1_hbm, out_hbm, idx_v, e0_v, e1_v, rows_v, y_v,
                      sem0, sem1):
    wid = lax.axis_index("s") * 2 + lax.axis_index("c")
    base = wid * _NPT
    boff = (base // _N) * _N
    sems = (sem0, sem1)

    def load_fire(ch, buf):
        g0 = base + ch * _NB
        pltpu.sync_copy(e0_hbm.at[pl.ds(g0 * _K, _NB * _K)], e0_v)
        pltpu.sync_copy(e1_hbm.at[pl.ds(g0 * _K, _NB * _K)], e1_v)
        o = buf * _ROWS

        def build(i, c2):
            sl = pl.ds(i * _K, _K)
            idx_v[pl.ds(o + i * 2 * _K, _K)] = e0_v[sl] + boff
            idx_v[pl.ds(o + i * 2 * _K + _K, _K)] = e1_v[sl] + boff
            return c2

        lax.fori_loop(0, _NB, build, 0)
        for r in range(_NGATHER):
            pltpu.async_copy(
                x_hbm.at[idx_v.at[pl.ds(o + r * _IDXCHUNK, _IDXCHUNK)]],
                rows_v.at[pl.ds(o + r * _IDXCHUNK, _IDXCHUNK)],
                sems[buf],
            )

    def wait_gathers(buf):
        o = buf * _ROWS
        for r in range(_NGATHER):
            pltpu.make_async_copy(
                x_hbm.at[idx_v.at[pl.ds(o + r * _IDXCHUNK, _IDXCHUNK)]],
                rows_v.at[pl.ds(o + r * _IDXCHUNK, _IDXCHUNK)],
                sems[buf],
            ).wait()

    def compute_store(ch, buf):
        g0 = base + ch * _NB
        o = buf * _ROWS

        def comp(i, c2):
            r0 = o + i * 2 * _K
            for cc in range(_CCB):
                sl = pl.ds(cc * _BLANES, _BLANES)
                acc = rows_v[r0, sl] - rows_v[r0 + _K, sl]
                for k in range(1, _K):
                    acc = jnp.maximum(acc, rows_v[r0 + k, sl] - rows_v[r0 + _K + k, sl])
                y_v[pl.ds(i * _C + cc * _BLANES, _BLANES)] = acc
            return c2

        lax.fori_loop(0, _NB, comp, 0)
        pltpu.sync_copy(y_v, out_hbm.at[pl.ds(g0 * _C, _NB * _C)])

    load_fire(0, 0)

    def outer(g, carry):
        for b in (0, 1):
            ch = g * 2 + b

            @pl.when(ch + 1 < _NCHUNK)
            def _():
                load_fire(ch + 1, b ^ 1)

            wait_gathers(b)
            compute_store(ch, b)
        return carry

    lax.fori_loop(0, _NCHUNK // 2, outer, 0)


@functools.cache
def _sc_edge_max():
    return pl.kernel(
        _sc_edge_max_body,
        out_type=jax.ShapeDtypeStruct((_B * _N * _C,), jnp.bfloat16),
        mesh=plsc.VectorSubcoreMesh(core_axis_name="c", subcore_axis_name="s"),
        compiler_params=pltpu.CompilerParams(use_tc_tiling_on_sc=False),
        scratch_types=[
            pltpu.VMEM((2 * _ROWS,), jnp.int32),
            pltpu.VMEM((_NB * _K,), jnp.int32),
            pltpu.VMEM((_NB * _K,), jnp.int32),
            pltpu.VMEM((2 * _ROWS, _C), jnp.bfloat16),
            pltpu.VMEM((_NB * _C,), jnp.bfloat16),
            pltpu.SemaphoreType.DMA,
            pltpu.SemaphoreType.DMA,
        ],
    )


def _tc_body(x_ref, xt_ref, ye_ref, m_ref, wxc_ref, we_ref, wc_ref, b_ref, o_ref):
    xb = x_ref[0]
    xt = xt_ref[0]
    ye = ye_ref[0].astype(jnp.float32)
    cent = jnp.dot(xb, m_ref[...], precision=_PREC)
    w = None
    for _ in range(3):
        xc = jnp.dot(xt, cent, precision=_PREC)
        c2 = jnp.sum(cent * cent, axis=0)[None, :]
        logits = 2.0 * xc - c2
        mx = jnp.max(logits, axis=1, keepdims=True)
        e = jnp.exp(logits - mx)
        w = e / jnp.sum(e, axis=1, keepdims=True)
        denom = jnp.sum(w, axis=0)[None, :] + 1e-8
        cent = jnp.dot(xb, w, precision=_PREC) / denom

    iota = lax.broadcasted_iota(jnp.int32, (_N, _NCENT), 1)
    avail = jnp.ones((_N, _NCENT), jnp.bool_)
    acc = None
    for _ in range(_TOPK):
        cur = jnp.where(avail, w, -1.0)
        mx = jnp.max(cur, axis=1, keepdims=True)
        ism = cur == mx
        minj = jnp.min(jnp.where(ism, iota, _NCENT), axis=1, keepdims=True)
        sel = iota == minj
        pick = lax.dot_general(sel.astype(jnp.float32), cent,
                               (((1,), (1,)), ((), ())), precision=_PREC)
        acc = pick if acc is None else jnp.maximum(acc, pick)
        avail = avail & jnp.logical_not(sel)

    out = lax.dot_general(xt, wxc_ref[...], (((1,), (1,)), ((), ())),
                          precision=_PREC)
    out = out + lax.dot_general(ye, we_ref[...], (((1,), (1,)), ((), ())),
                                precision=_PREC)
    out = out + lax.dot_general(acc, wc_ref[...], (((1,), (1,)), ((), ())),
                                precision=_PREC)
    out = out + b_ref[...]
    o_ref[0] = jnp.maximum(out, 0.0)


def kernel(x, edge_index, W_conv, b_conv, H, W):
    xb = x[..., 0]
    xt = jnp.swapaxes(xb, 1, 2)
    x_rows = xt.reshape(_B * _N, _C).astype(jnp.bfloat16)
    ei = edge_index.astype(jnp.int32)
    e0 = ei[0].reshape(-1)
    e1 = ei[1].reshape(-1)
    y_edge = _sc_edge_max()(x_rows, e0, e1)
    ye = y_edge.reshape(_B, _N, _C)

    w3 = W_conv[:, :, 0, 0].reshape(_C, _C // _G, 3)
    gmask = jnp.asarray(_GMASK)

    def expand(ws):
        return jnp.tile(ws, (1, _G)) * gmask

    Wx = expand(w3[:, :, 0])
    We = expand(w3[:, :, 1])
    Wc = expand(w3[:, :, 2])
    Wxc = Wx - Wc
    b2 = b_conv.reshape(1, _C)

    out = pl.pallas_call(
        _tc_body,
        grid=(_B,),
        in_specs=[
            pl.BlockSpec((1, _C, _N), lambda b: (b, 0, 0)),
            pl.BlockSpec((1, _N, _C), lambda b: (b, 0, 0)),
            pl.BlockSpec((1, _N, _C), lambda b: (b, 0, 0)),
            pl.BlockSpec((_N, _NCENT), lambda b: (0, 0)),
            pl.BlockSpec((_C, _C), lambda b: (0, 0)),
            pl.BlockSpec((_C, _C), lambda b: (0, 0)),
            pl.BlockSpec((_C, _C), lambda b: (0, 0)),
            pl.BlockSpec((1, _C), lambda b: (0, 0)),
        ],
        out_specs=pl.BlockSpec((1, _N, _C), lambda b: (b, 0, 0)),
        out_shape=jax.ShapeDtypeStruct((_B, _N, _C), jnp.float32),
    )(xb, xt, ye, jnp.asarray(_M_POOL), Wxc, We, Wc, b2)
    return jnp.swapaxes(out, 1, 2)[..., None]


# speedup vs baseline: 3954.6936x; 1.0479x over previous
import functools

import numpy as np
import jax
import jax.numpy as jnp
from jax import lax
from jax.experimental import pallas as pl
from jax.experimental.pallas import tpu as pltpu
from jax.experimental.pallas import tpu_sc as plsc

_B, _C, _N, _K = 4, 96, 2304, 16
_NCENT, _TOPK, _G = 50, 12, 4
_PREC = None

_NTILES = 32
_NPT = (_B * _N) // _NTILES
_NB = 16
_NCHUNK = _NPT // _NB
_ROWS = _NB * 2 * _K
_IDXCHUNK = 128
_NGATHER = _ROWS // _IDXCHUNK
_LANES = 16
_BLANES = 32
_CCB = _C // _BLANES


def _pool_matrix() -> np.ndarray:
    Hs = Ws = 48
    oh, ow = 5, 10
    M = np.zeros((Hs * Ws, oh * ow), np.float32)
    for i in range(oh):
        h0 = (i * Hs) // oh
        h1 = -(-((i + 1) * Hs) // oh)
        for j in range(ow):
            w0 = (j * Ws) // ow
            w1 = -(-((j + 1) * Ws) // ow)
            cnt = (h1 - h0) * (w1 - w0)
            for h in range(h0, h1):
                for w in range(w0, w1):
                    M[h * Ws + w, i * ow + j] = 1.0 / cnt
    return M


_M_POOL = _pool_matrix()
_GMASK = ((np.arange(_C)[:, None] // (_C // _G))
          == (np.arange(_C)[None, :] // (_C // _G))).astype(np.float32)


def _sc_edge_max_body(x_hbm, e0_hbm, e1_hbm, out_hbm, idx_v, e0_v, e1_v, rows_v, y_v,
                      sem0, sem1, semw0, semw1):
    wid = lax.axis_index("s") * 2 + lax.axis_index("c")
    base = wid * _NPT
    boff = (base // _N) * _N
    sems = (sem0, sem1)
    semws = (semw0, semw1)

    pltpu.sync_copy(e0_hbm.at[pl.ds(base * _K, _NPT * _K)], e0_v)
    pltpu.sync_copy(e1_hbm.at[pl.ds(base * _K, _NPT * _K)], e1_v)

    def build(i, c2):
        sl = pl.ds(i * _K, _K)
        idx_v[pl.ds(i * 2 * _K, _K)] = e0_v[sl] + boff
        idx_v[pl.ds(i * 2 * _K + _K, _K)] = e1_v[sl] + boff
        return c2

    lax.fori_loop(0, _NPT, build, 0)

    def fire_gathers(ch, buf):
        o = buf * _ROWS
        for r in range(_NGATHER):
            pltpu.async_copy(
                x_hbm.at[idx_v.at[pl.ds(ch * _ROWS + r * _IDXCHUNK, _IDXCHUNK)]],
                rows_v.at[pl.ds(o + r * _IDXCHUNK, _IDXCHUNK)],
                sems[buf],
            )

    def wait_gathers(ch, buf):
        o = buf * _ROWS
        for r in range(_NGATHER):
            pltpu.make_async_copy(
                x_hbm.at[idx_v.at[pl.ds(ch * _ROWS + r * _IDXCHUNK, _IDXCHUNK)]],
                rows_v.at[pl.ds(o + r * _IDXCHUNK, _IDXCHUNK)],
                sems[buf],
            ).wait()

    def wb_copy(ch, buf):
        g0 = base + ch * _NB
        return pltpu.make_async_copy(
            y_v.at[pl.ds(buf * _NB * _C, _NB * _C)],
            out_hbm.at[pl.ds(g0 * _C, _NB * _C)],
            semws[buf],
        )

    def compute(ch, buf):
        o = buf * _ROWS
        yo = buf * _NB * _C

        def comp(i, c2):
            r0 = o + i * 2 * _K
            for cc in range(_CCB):
                sl = pl.ds(cc * _BLANES, _BLANES)
                acc = rows_v[r0, sl] - rows_v[r0 + _K, sl]
                for k in range(1, _K):
                    acc = jnp.maximum(acc, rows_v[r0 + k, sl] - rows_v[r0 + _K + k, sl])
                y_v[pl.ds(yo + i * _C + cc * _BLANES, _BLANES)] = acc
            return c2

        lax.fori_loop(0, _NB, comp, 0)

    fire_gathers(0, 0)

    def outer(g, carry):
        for b in (0, 1):
            ch = g * 2 + b

            @pl.when(ch + 1 < _NCHUNK)
            def _():
                fire_gathers(ch + 1, b ^ 1)

            wait_gathers(ch, b)

            @pl.when(ch >= 2)
            def _():
                wb_copy(ch - 2, b).wait()

            compute(ch, b)
            wb_copy(ch, b).start()
        return carry

    lax.fori_loop(0, _NCHUNK // 2, outer, 0)
    wb_copy(_NCHUNK - 2, 0).wait()
    wb_copy(_NCHUNK - 1, 1).wait()


@functools.cache
def _sc_edge_max():
    return pl.kernel(
        _sc_edge_max_body,
        out_type=jax.ShapeDtypeStruct((_B * _N * _C,), jnp.bfloat16),
        mesh=plsc.VectorSubcoreMesh(core_axis_name="c", subcore_axis_name="s"),
        compiler_params=pltpu.CompilerParams(use_tc_tiling_on_sc=False),
        scratch_types=[
            pltpu.VMEM((_NPT * 2 * _K,), jnp.int32),
            pltpu.VMEM((_NPT * _K,), jnp.int32),
            pltpu.VMEM((_NPT * _K,), jnp.int32),
            pltpu.VMEM((2 * _ROWS, _C), jnp.bfloat16),
            pltpu.VMEM((2 * _NB * _C,), jnp.bfloat16),
            pltpu.SemaphoreType.DMA,
            pltpu.SemaphoreType.DMA,
            pltpu.SemaphoreType.DMA,
            pltpu.SemaphoreType.DMA,
        ],
    )


def _tc_body(x_ref, xt_ref, ye_ref, m_ref, wxc_ref, we_ref, wc_ref, b_ref, o_ref):
    xb = x_ref[0]
    xt = xt_ref[0]
    ye = ye_ref[0].astype(jnp.float32)
    cent = jnp.dot(xb, m_ref[...], precision=_PREC)
    w = None
    for _ in range(3):
        xc = jnp.dot(xt, cent, precision=_PREC)
        c2 = jnp.sum(cent * cent, axis=0)[None, :]
        logits = 2.0 * xc - c2
        mx = jnp.max(logits, axis=1, keepdims=True)
        e = jnp.exp(logits - mx)
        w = e / jnp.sum(e, axis=1, keepdims=True)
        denom = jnp.sum(w, axis=0)[None, :] + 1e-8
        cent = jnp.dot(xb, w, precision=_PREC) / denom

    iota = lax.broadcasted_iota(jnp.int32, (_N, _NCENT), 1)
    avail = jnp.ones((_N, _NCENT), jnp.bool_)
    acc = None
    for _ in range(_TOPK):
        cur = jnp.where(avail, w, -1.0)
        mx = jnp.max(cur, axis=1, keepdims=True)
        ism = cur == mx
        minj = jnp.min(jnp.where(ism, iota, _NCENT), axis=1, keepdims=True)
        sel = iota == minj
        pick = lax.dot_general(sel.astype(jnp.float32), cent,
                               (((1,), (1,)), ((), ())), precision=_PREC)
        acc = pick if acc is None else jnp.maximum(acc, pick)
        avail = avail & jnp.logical_not(sel)

    out = lax.dot_general(xt, wxc_ref[...], (((1,), (1,)), ((), ())),
                          precision=_PREC)
    out = out + lax.dot_general(ye, we_ref[...], (((1,), (1,)), ((), ())),
                                precision=_PREC)
    out = out + lax.dot_general(acc, wc_ref[...], (((1,), (1,)), ((), ())),
                                precision=_PREC)
    out = out + b_ref[...]
    o_ref[0] = jnp.maximum(out, 0.0)


def kernel(x, edge_index, W_conv, b_conv, H, W):
    xb = x[..., 0]
    xt = jnp.swapaxes(xb, 1, 2)
    x_rows = xt.reshape(_B * _N, _C).astype(jnp.bfloat16)
    ei = edge_index.astype(jnp.int32)
    e0 = ei[0].reshape(-1)
    e1 = ei[1].reshape(-1)
    y_edge = _sc_edge_max()(x_rows, e0, e1)
    ye = y_edge.reshape(_B, _N, _C)

    w3 = W_conv[:, :, 0, 0].reshape(_C, _C // _G, 3)
    gmask = jnp.asarray(_GMASK)

    def expand(ws):
        return jnp.tile(ws, (1, _G)) * gmask

    Wx = expand(w3[:, :, 0])
    We = expand(w3[:, :, 1])
    Wc = expand(w3[:, :, 2])
    Wxc = Wx - Wc
    b2 = b_conv.reshape(1, _C)

    out = pl.pallas_call(
        _tc_body,
        grid=(_B,),
        in_specs=[
            pl.BlockSpec((1, _C, _N), lambda b: (b, 0, 0)),
            pl.BlockSpec((1, _N, _C), lambda b: (b, 0, 0)),
            pl.BlockSpec((1, _N, _C), lambda b: (b, 0, 0)),
            pl.BlockSpec((_N, _NCENT), lambda b: (0, 0)),
            pl.BlockSpec((_C, _C), lambda b: (0, 0)),
            pl.BlockSpec((_C, _C), lambda b: (0, 0)),
            pl.BlockSpec((_C, _C), lambda b: (0, 0)),
            pl.BlockSpec((1, _C), lambda b: (0, 0)),
        ],
        out_specs=pl.BlockSpec((1, _N, _C), lambda b: (b, 0, 0)),
        out_shape=jax.ShapeDtypeStruct((_B, _N, _C), jnp.float32),
    )(xb, xt, ye, jnp.asarray(_M_POOL), Wxc, We, Wc, b2)
    return jnp.swapaxes(out, 1, 2)[..., None]


# speedup vs baseline: 4622.1551x; 1.1688x over previous
import functools

import numpy as np
import jax
import jax.numpy as jnp
from jax import lax
from jax.experimental import pallas as pl
from jax.experimental.pallas import tpu as pltpu
from jax.experimental.pallas import tpu_sc as plsc

_B, _C, _N, _K = 4, 96, 2304, 16
_NCENT, _TOPK, _G = 50, 12, 4
_PREC = None

_NTILES = 32
_NPT = (_B * _N) // _NTILES
_NB = 16
_NCHUNK = _NPT // _NB
_ROWS = _NB * 2 * _K
_IDXCHUNK = 128
_NGATHER = _ROWS // _IDXCHUNK
_LANES = 16
_BLANES = 32
_CCB = _C // _BLANES


def _pool_matrix() -> np.ndarray:
    Hs = Ws = 48
    oh, ow = 5, 10
    M = np.zeros((Hs * Ws, oh * ow), np.float32)
    for i in range(oh):
        h0 = (i * Hs) // oh
        h1 = -(-((i + 1) * Hs) // oh)
        for j in range(ow):
            w0 = (j * Ws) // ow
            w1 = -(-((j + 1) * Ws) // ow)
            cnt = (h1 - h0) * (w1 - w0)
            for h in range(h0, h1):
                for w in range(w0, w1):
                    M[h * Ws + w, i * ow + j] = 1.0 / cnt
    return M


_M_POOL = _pool_matrix()
_GMASK = ((np.arange(_C)[:, None] // (_C // _G))
          == (np.arange(_C)[None, :] // (_C // _G))).astype(np.float32)


def _sc_edge_max_body(x_hbm, e0_hbm, e1_hbm, out_hbm, idx_v, e0_v, e1_v, rows_v, y_v,
                      sem0, sem1, semw0, semw1):
    wid = lax.axis_index("s") * 2 + lax.axis_index("c")
    base = wid * _NPT
    boff = (base // _N) * _N
    sems = (sem0, sem1)
    semws = (semw0, semw1)

    pltpu.sync_copy(e0_hbm.at[pl.ds(base * _K, _NPT * _K)], e0_v)
    pltpu.sync_copy(e1_hbm.at[pl.ds(base * _K, _NPT * _K)], e1_v)

    def build(i, c2):
        sl = pl.ds(i * _K, _K)
        idx_v[pl.ds(i * 2 * _K, _K)] = e0_v[sl] + boff
        idx_v[pl.ds(i * 2 * _K + _K, _K)] = e1_v[sl] + boff
        return c2

    lax.fori_loop(0, _NPT, build, 0)

    def fire_gathers(ch, buf):
        o = buf * _ROWS
        for r in range(_NGATHER):
            pltpu.async_copy(
                x_hbm.at[idx_v.at[pl.ds(ch * _ROWS + r * _IDXCHUNK, _IDXCHUNK)]],
                rows_v.at[pl.ds(o + r * _IDXCHUNK, _IDXCHUNK)],
                sems[buf],
            )

    def wait_gathers(ch, buf):
        o = buf * _ROWS
        for r in range(_NGATHER):
            pltpu.make_async_copy(
                x_hbm.at[idx_v.at[pl.ds(ch * _ROWS + r * _IDXCHUNK, _IDXCHUNK)]],
                rows_v.at[pl.ds(o + r * _IDXCHUNK, _IDXCHUNK)],
                sems[buf],
            ).wait()

    def wb_copy(ch, buf):
        g0 = base + ch * _NB
        return pltpu.make_async_copy(
            y_v.at[pl.ds(buf * _NB * _C, _NB * _C)],
            out_hbm.at[pl.ds(g0 * _C, _NB * _C)],
            semws[buf],
        )

    def compute(ch, buf):
        o = buf * _ROWS
        yo = buf * _NB * _C

        def comp(i, c2):
            r0 = o + i * 2 * _K
            for cc in range(_CCB):
                sl = pl.ds(cc * _BLANES, _BLANES)
                acc = rows_v[r0, sl] - rows_v[r0 + _K, sl]
                for k in range(1, _K):
                    acc = jnp.maximum(acc, rows_v[r0 + k, sl] - rows_v[r0 + _K + k, sl])
                y_v[pl.ds(yo + i * _C + cc * _BLANES, _BLANES)] = acc
            return c2

        lax.fori_loop(0, _NB, comp, 0)

    fire_gathers(0, 0)

    def outer(g, carry):
        for b in (0, 1):
            ch = g * 2 + b

            @pl.when(ch + 1 < _NCHUNK)
            def _():
                fire_gathers(ch + 1, b ^ 1)

            wait_gathers(ch, b)

            @pl.when(ch >= 2)
            def _():
                wb_copy(ch - 2, b).wait()

            compute(ch, b)
            wb_copy(ch, b).start()
        return carry

    lax.fori_loop(0, _NCHUNK // 2, outer, 0)
    wb_copy(_NCHUNK - 2, 0).wait()
    wb_copy(_NCHUNK - 1, 1).wait()


@functools.cache
def _sc_edge_max():
    return pl.kernel(
        _sc_edge_max_body,
        out_type=jax.ShapeDtypeStruct((_B * _N * _C,), jnp.bfloat16),
        mesh=plsc.VectorSubcoreMesh(core_axis_name="c", subcore_axis_name="s"),
        compiler_params=pltpu.CompilerParams(use_tc_tiling_on_sc=False),
        scratch_types=[
            pltpu.VMEM((_NPT * 2 * _K,), jnp.int32),
            pltpu.VMEM((_NPT * _K,), jnp.int32),
            pltpu.VMEM((_NPT * _K,), jnp.int32),
            pltpu.VMEM((2 * _ROWS, _C), jnp.bfloat16),
            pltpu.VMEM((2 * _NB * _C,), jnp.bfloat16),
            pltpu.SemaphoreType.DMA,
            pltpu.SemaphoreType.DMA,
            pltpu.SemaphoreType.DMA,
            pltpu.SemaphoreType.DMA,
        ],
    )


def _tc_centers_body(x_ref, xt_ref, m_ref, o_ref):
    xb = x_ref[0]
    xt = xt_ref[0]
    cent = jnp.dot(xb, m_ref[...], precision=_PREC)
    w = None
    for _ in range(3):
        xc = jnp.dot(xt, cent, precision=_PREC)
        c2 = jnp.sum(cent * cent, axis=0)[None, :]
        logits = 2.0 * xc - c2
        mx = jnp.max(logits, axis=1, keepdims=True)
        e = jnp.exp(logits - mx)
        w = e / jnp.sum(e, axis=1, keepdims=True)
        denom = jnp.sum(w, axis=0)[None, :] + 1e-8
        cent = jnp.dot(xb, w, precision=_PREC) / denom

    iota = lax.broadcasted_iota(jnp.int32, (_N, _NCENT), 1)
    avail = jnp.ones((_N, _NCENT), jnp.bool_)
    acc = None
    for _ in range(_TOPK):
        cur = jnp.where(avail, w, -1.0)
        mx = jnp.max(cur, axis=1, keepdims=True)
        ism = cur == mx
        minj = jnp.min(jnp.where(ism, iota, _NCENT), axis=1, keepdims=True)
        sel = iota == minj
        pick = lax.dot_general(sel.astype(jnp.float32), cent,
                               (((1,), (1,)), ((), ())), precision=_PREC)
        acc = pick if acc is None else jnp.maximum(acc, pick)
        avail = avail & jnp.logical_not(sel)

    o_ref[0] = acc


def _tc_conv_body(xt_ref, ye_ref, acc_ref, wxc_ref, we_ref, wc_ref, b_ref, o_ref):
    xt = xt_ref[0]
    ye = ye_ref[0].astype(jnp.float32)
    acc = acc_ref[0]
    out = lax.dot_general(xt, wxc_ref[...], (((1,), (1,)), ((), ())),
                          precision=_PREC)
    out = out + lax.dot_general(ye, we_ref[...], (((1,), (1,)), ((), ())),
                                precision=_PREC)
    out = out + lax.dot_general(acc, wc_ref[...], (((1,), (1,)), ((), ())),
                                precision=_PREC)
    out = out + b_ref[...]
    o_ref[0] = jnp.maximum(out, 0.0)


def kernel(x, edge_index, W_conv, b_conv, H, W):
    xb = x[..., 0]
    xt = jnp.swapaxes(xb, 1, 2)
    x_rows = xt.reshape(_B * _N, _C).astype(jnp.bfloat16)
    ei = edge_index.astype(jnp.int32)
    e0 = ei[0].reshape(-1)
    e1 = ei[1].reshape(-1)
    y_edge = _sc_edge_max()(x_rows, e0, e1)
    ye = y_edge.reshape(_B, _N, _C)

    w3 = W_conv[:, :, 0, 0].reshape(_C, _C // _G, 3)
    gmask = jnp.asarray(_GMASK)

    def expand(ws):
        return jnp.tile(ws, (1, _G)) * gmask

    Wx = expand(w3[:, :, 0])
    We = expand(w3[:, :, 1])
    Wc = expand(w3[:, :, 2])
    Wxc = Wx - Wc
    b2 = b_conv.reshape(1, _C)

    acc = pl.pallas_call(
        _tc_centers_body,
        grid=(_B,),
        in_specs=[
            pl.BlockSpec((1, _C, _N), lambda b: (b, 0, 0)),
            pl.BlockSpec((1, _N, _C), lambda b: (b, 0, 0)),
            pl.BlockSpec((_N, _NCENT), lambda b: (0, 0)),
        ],
        out_specs=pl.BlockSpec((1, _N, _C), lambda b: (b, 0, 0)),
        out_shape=jax.ShapeDtypeStruct((_B, _N, _C), jnp.float32),
    )(xb, xt, jnp.asarray(_M_POOL))

    out = pl.pallas_call(
        _tc_conv_body,
        grid=(_B,),
        in_specs=[
            pl.BlockSpec((1, _N, _C), lambda b: (b, 0, 0)),
            pl.BlockSpec((1, _N, _C), lambda b: (b, 0, 0)),
            pl.BlockSpec((1, _N, _C), lambda b: (b, 0, 0)),
            pl.BlockSpec((_C, _C), lambda b: (0, 0)),
            pl.BlockSpec((_C, _C), lambda b: (0, 0)),
            pl.BlockSpec((_C, _C), lambda b: (0, 0)),
            pl.BlockSpec((1, _C), lambda b: (0, 0)),
        ],
        out_specs=pl.BlockSpec((1, _N, _C), lambda b: (b, 0, 0)),
        out_shape=jax.ShapeDtypeStruct((_B, _N, _C), jnp.float32),
    )(xt, ye, acc, Wxc, We, Wc, b2)
    return jnp.swapaxes(out, 1, 2)[..., None]


# speedup vs baseline: 4695.0288x; 1.0158x over previous
import functools

import numpy as np
import jax
import jax.numpy as jnp
from jax import lax
from jax.experimental import pallas as pl
from jax.experimental.pallas import tpu as pltpu
from jax.experimental.pallas import tpu_sc as plsc

_B, _C, _N, _K = 4, 96, 2304, 16
_NCENT, _TOPK, _G = 50, 12, 4
_PREC = None

_NTILES = 32
_NPT = (_B * _N) // _NTILES
_NB = 16
_NCHUNK = _NPT // _NB
_ROWS = _NB * 2 * _K
_IDXCHUNK = 128
_NGATHER = _ROWS // _IDXCHUNK
_LANES = 16
_BLANES = 32
_CCB = _C // _BLANES


def _pool_matrix() -> np.ndarray:
    Hs = Ws = 48
    oh, ow = 5, 10
    M = np.zeros((Hs * Ws, oh * ow), np.float32)
    for i in range(oh):
        h0 = (i * Hs) // oh
        h1 = -(-((i + 1) * Hs) // oh)
        for j in range(ow):
            w0 = (j * Ws) // ow
            w1 = -(-((j + 1) * Ws) // ow)
            cnt = (h1 - h0) * (w1 - w0)
            for h in range(h0, h1):
                for w in range(w0, w1):
                    M[h * Ws + w, i * ow + j] = 1.0 / cnt
    return M


_M_POOL = _pool_matrix()
_GMASK = ((np.arange(_C)[:, None] // (_C // _G))
          == (np.arange(_C)[None, :] // (_C // _G))).astype(np.float32)


def _sc_edge_max_body(x_hbm, e0_hbm, e1_hbm, out_hbm, idx_v, e0_v, e1_v, rows_v, y_v,
                      sem0, sem1, semw0, semw1):
    wid = lax.axis_index("s") * 2 + lax.axis_index("c")
    base = wid * _NPT
    boff = (base // _N) * _N
    sems = (sem0, sem1)
    semws = (semw0, semw1)

    pltpu.sync_copy(e0_hbm.at[pl.ds(base * _K, _NPT * _K)], e0_v)
    pltpu.sync_copy(e1_hbm.at[pl.ds(base * _K, _NPT * _K)], e1_v)

    def build(i, c2):
        sl = pl.ds(i * _K, _K)
        idx_v[pl.ds(i * 2 * _K, _K)] = e0_v[sl] + boff
        idx_v[pl.ds(i * 2 * _K + _K, _K)] = e1_v[sl] + boff
        return c2

    lax.fori_loop(0, _NPT, build, 0)

    def fire_gathers(ch, buf):
        o = buf * _ROWS
        for r in range(_NGATHER):
            pltpu.async_copy(
                x_hbm.at[idx_v.at[pl.ds(ch * _ROWS + r * _IDXCHUNK, _IDXCHUNK)]],
                rows_v.at[pl.ds(o + r * _IDXCHUNK, _IDXCHUNK)],
                sems[buf],
            )

    def wait_gathers(ch, buf):
        o = buf * _ROWS
        for r in range(_NGATHER):
            pltpu.make_async_copy(
                x_hbm.at[idx_v.at[pl.ds(ch * _ROWS + r * _IDXCHUNK, _IDXCHUNK)]],
                rows_v.at[pl.ds(o + r * _IDXCHUNK, _IDXCHUNK)],
                sems[buf],
            ).wait()

    def wb_copy(ch, buf):
        g0 = base + ch * _NB
        return pltpu.make_async_copy(
            y_v.at[pl.ds(buf * _NB * _C, _NB * _C)],
            out_hbm.at[pl.ds(g0 * _C, _NB * _C)],
            semws[buf],
        )

    def compute(ch, buf):
        o = buf * _ROWS
        yo = buf * _NB * _C

        def comp(i, c2):
            r0 = o + i * 2 * _K
            for cc in range(_CCB):
                sl = pl.ds(cc * _BLANES, _BLANES)
                acc = rows_v[r0, sl] - rows_v[r0 + _K, sl]
                for k in range(1, _K):
                    acc = jnp.maximum(acc, rows_v[r0 + k, sl] - rows_v[r0 + _K + k, sl])
                y_v[pl.ds(yo + i * _C + cc * _BLANES, _BLANES)] = acc
            return c2

        lax.fori_loop(0, _NB, comp, 0)

    fire_gathers(0, 0)

    def outer(g, carry):
        for b in (0, 1):
            ch = g * 2 + b

            @pl.when(ch + 1 < _NCHUNK)
            def _():
                fire_gathers(ch + 1, b ^ 1)

            wait_gathers(ch, b)

            @pl.when(ch >= 2)
            def _():
                wb_copy(ch - 2, b).wait()

            compute(ch, b)
            wb_copy(ch, b).start()
        return carry

    lax.fori_loop(0, _NCHUNK // 2, outer, 0)
    wb_copy(_NCHUNK - 2, 0).wait()
    wb_copy(_NCHUNK - 1, 1).wait()


@functools.cache
def _sc_edge_max():
    return pl.kernel(
        _sc_edge_max_body,
        out_type=jax.ShapeDtypeStruct((_B * _N * _C,), jnp.bfloat16),
        mesh=plsc.VectorSubcoreMesh(core_axis_name="c", subcore_axis_name="s"),
        compiler_params=pltpu.CompilerParams(use_tc_tiling_on_sc=False),
        scratch_types=[
            pltpu.VMEM((_NPT * 2 * _K,), jnp.int32),
            pltpu.VMEM((_NPT * _K,), jnp.int32),
            pltpu.VMEM((_NPT * _K,), jnp.int32),
            pltpu.VMEM((2 * _ROWS, _C), jnp.bfloat16),
            pltpu.VMEM((2 * _NB * _C,), jnp.bfloat16),
            pltpu.SemaphoreType.DMA,
            pltpu.SemaphoreType.DMA,
            pltpu.SemaphoreType.DMA,
            pltpu.SemaphoreType.DMA,
        ],
    )


def _tc_centers_body(x_ref, xt_ref, m_ref, o_ref):
    xb = x_ref[0]
    xt = xt_ref[0]
    cent = jnp.dot(xb, m_ref[...], precision=_PREC)
    w = None
    for _ in range(3):
        xc = jnp.dot(xt, cent, precision=_PREC)
        c2 = jnp.sum(cent * cent, axis=0)[None, :]
        logits = 2.0 * xc - c2
        mx = jnp.max(logits, axis=1, keepdims=True)
        e = jnp.exp(logits - mx)
        w = e / jnp.sum(e, axis=1, keepdims=True)
        denom = jnp.sum(w, axis=0)[None, :] + 1e-8
        cent = jnp.dot(xb, w, precision=_PREC) / denom

    iota = lax.broadcasted_iota(jnp.int32, (_N, _NCENT), 1)
    avail = jnp.ones((_N, _NCENT), jnp.bool_)
    acc = None
    for _ in range(_TOPK):
        cur = jnp.where(avail, w, -1.0)
        mx = jnp.max(cur, axis=1, keepdims=True)
        ism = cur == mx
        minj = jnp.min(jnp.where(ism, iota, _NCENT), axis=1, keepdims=True)
        sel = iota == minj
        pick = lax.dot_general(sel.astype(jnp.float32), cent,
                               (((1,), (1,)), ((), ())), precision=_PREC)
        acc = pick if acc is None else jnp.maximum(acc, pick)
        avail = avail & jnp.logical_not(sel)

    o_ref[0] = acc


def _tc_conv_body(xt_ref, ye_ref, acc_ref, wxc_ref, we_ref, wc_ref, b_ref, o_ref):
    xt = xt_ref[0]
    ye = ye_ref[0].astype(jnp.float32)
    acc = acc_ref[0]
    out = lax.dot_general(xt, wxc_ref[...], (((1,), (1,)), ((), ())),
                          precision=_PREC)
    out = out + lax.dot_general(ye, we_ref[...], (((1,), (1,)), ((), ())),
                                precision=_PREC)
    out = out + lax.dot_general(acc, wc_ref[...], (((1,), (1,)), ((), ())),
                                precision=_PREC)
    out = out + b_ref[...]
    o_ref[0] = lax.transpose(jnp.maximum(out, 0.0), (1, 0))


def kernel(x, edge_index, W_conv, b_conv, H, W):
    xb = x[..., 0]
    xt = jnp.swapaxes(xb, 1, 2)
    x_rows = xt.reshape(_B * _N, _C).astype(jnp.bfloat16)
    ei = edge_index.astype(jnp.int32)
    e0 = ei[0].reshape(-1)
    e1 = ei[1].reshape(-1)
    y_edge = _sc_edge_max()(x_rows, e0, e1)
    ye = y_edge.reshape(_B, _N, _C)

    w3 = W_conv[:, :, 0, 0].reshape(_C, _C // _G, 3)
    gmask = jnp.asarray(_GMASK)

    def expand(ws):
        return jnp.tile(ws, (1, _G)) * gmask

    Wx = expand(w3[:, :, 0])
    We = expand(w3[:, :, 1])
    Wc = expand(w3[:, :, 2])
    Wxc = Wx - Wc
    b2 = b_conv.reshape(1, _C)

    acc = pl.pallas_call(
        _tc_centers_body,
        grid=(_B,),
        in_specs=[
            pl.BlockSpec((1, _C, _N), lambda b: (b, 0, 0)),
            pl.BlockSpec((1, _N, _C), lambda b: (b, 0, 0)),
            pl.BlockSpec((_N, _NCENT), lambda b: (0, 0)),
        ],
        out_specs=pl.BlockSpec((1, _N, _C), lambda b: (b, 0, 0)),
        out_shape=jax.ShapeDtypeStruct((_B, _N, _C), jnp.float32),
    )(xb, xt, jnp.asarray(_M_POOL))

    out = pl.pallas_call(
        _tc_conv_body,
        grid=(_B,),
        in_specs=[
            pl.BlockSpec((1, _N, _C), lambda b: (b, 0, 0)),
            pl.BlockSpec((1, _N, _C), lambda b: (b, 0, 0)),
            pl.BlockSpec((1, _N, _C), lambda b: (b, 0, 0)),
            pl.BlockSpec((_C, _C), lambda b: (0, 0)),
            pl.BlockSpec((_C, _C), lambda b: (0, 0)),
            pl.BlockSpec((_C, _C), lambda b: (0, 0)),
            pl.BlockSpec((1, _C), lambda b: (0, 0)),
        ],
        out_specs=pl.BlockSpec((1, _C, _N), lambda b: (b, 0, 0)),
        out_shape=jax.ShapeDtypeStruct((_B, _C, _N), jnp.float32),
    )(xt, ye, acc, Wxc, We, Wc, b2)
    return out[..., None]
